# baseline (speedup 1.0000x reference)
import functools

import jax
import jax.numpy as jnp
from jax import lax
from jax.experimental import pallas as pl
from jax.experimental.pallas import tpu as pltpu
from jax.experimental.pallas import tpu_sc as plsc

B = 16384
D = 128
L = 16
NC = 2
NS = 16
NW = NC * NS
SB = 8192
BPW = SB // NW
CH = 128
NCHUNK = BPW // CH
NREL = 1000
NRELP = 1024
BLK = 512
MT = B - SB
NB = MT // BLK

_mesh = plsc.VectorSubcoreMesh(core_axis_name="c", subcore_axis_name="s")


@functools.partial(
    pl.kernel,
    out_type=jax.ShapeDtypeStruct((SB,), jnp.float32),
    mesh=_mesh,
    compiler_params=pltpu.CompilerParams(needs_layout_passes=False),
    scratch_types=[
        pltpu.VMEM((BPW,), jnp.int32),
        pltpu.VMEM((BPW,), jnp.float32),
        pltpu.VMEM((2, CH, D), jnp.float32),
        pltpu.VMEM((2, CH, D), jnp.float32),
        pltpu.VMEM((2, CH, D), jnp.float32),
        pltpu.SemaphoreType.DMA,
        pltpu.SemaphoreType.DMA,
    ],
)
def _distmult_sc(h_hbm, r_hbm, t_hbm, rel_hbm, out_hbm,
                 idx_v, o_v, h_b, t_b, rel_b, sem0, sem1):
    wid = lax.axis_index("s") * NC + lax.axis_index("c")
    base = wid * BPW

    lane = lax.iota(jnp.int32, L)
    perms = [lane ^ s for s in (8, 4, 2, 1)]
    lane0 = lane == 0
    sems = (sem0, sem1)

    def start_ht(c):
        k = c & 1
        cbase = base + c * CH
        return (
            pltpu.async_copy(h_hbm.at[pl.ds(cbase, CH), :], h_b.at[k], sems[k]),
            pltpu.async_copy(t_hbm.at[pl.ds(cbase, CH), :], t_b.at[k], sems[k]),
        )

    def start_g(c):
        k = c & 1
        return (
            pltpu.async_copy(rel_hbm.at[idx_v.at[pl.ds(c * CH, CH)]],
                             rel_b.at[k], sems[k]),
        )

    pend = start_ht(0)
    pltpu.sync_copy(r_hbm.at[pl.ds(base, BPW)], idx_v)
    pend = pend + start_g(0)

    for c in range(NCHUNK):
        nxt = (start_ht(c + 1) + start_g(c + 1)) if c + 1 < NCHUNK else None
        for dsc in pend:
            dsc.wait()
        k = c & 1
        hk, tk, rk = h_b.at[k], t_b.at[k], rel_b.at[k]
        obase = c * CH

        def row(i, _):
            acc = hk[i, pl.ds(0, L)] * rk[i, pl.ds(0, L)] * tk[i, pl.ds(0, L)]
            for j in range(1, D // L):
                sl = pl.ds(j * L, L)
                acc = acc + hk[i, sl] * rk[i, sl] * tk[i, sl]
            for pm in perms:
                acc = acc + acc.at[pm].get(mode="promise_in_bounds")
            plsc.store_scatter(o_v, [jnp.full((L,), obase + i, jnp.int32)],
                               acc, mask=lane0)
            return 0

        lax.fori_loop(0, CH, row, 0, unroll=2)
        pend = nxt

    pltpu.sync_copy(o_v, out_hbm.at[pl.ds(base, BPW)])


def _tc_body(rel_ref, r_ref, h_ref, t_ref, out_ref):
    r2 = r_ref[0]
    k_iota = lax.broadcasted_iota(jnp.int32, (BLK, NRELP), 1)
    oh = (k_iota == r2).astype(jnp.bfloat16)
    g = lax.dot_general(oh, rel_ref[...], (((1,), (0,)), ((), ())),
                        preferred_element_type=jnp.float32)
    s = jnp.sum(h_ref[...] * g * t_ref[...], axis=1, keepdims=True)
    out_ref[...] = s


_tc_call = pl.pallas_call(
    _tc_body,
    grid=(NB,),
    in_specs=[
        pl.BlockSpec((NRELP, D), lambda i: (0, 0)),
        pl.BlockSpec((1, BLK, 1), lambda i: (i, 0, 0)),
        pl.BlockSpec((BLK, D), lambda i: (i, 0)),
        pl.BlockSpec((BLK, D), lambda i: (i, 0)),
    ],
    out_specs=pl.BlockSpec((BLK, 1), lambda i: (i, 0)),
    out_shape=jax.ShapeDtypeStruct((MT, 1), jnp.float32),
)


def kernel(h, r, t, mode, rel_emb):
    del mode
    r32 = r.astype(jnp.int32)
    out_sc = _distmult_sc(h[:SB], r32[:SB], t[:SB], rel_emb)
    relp = jnp.zeros((NRELP, D), jnp.bfloat16).at[:NREL].set(
        rel_emb.astype(jnp.bfloat16))
    rt = r32[SB:].reshape(NB, BLK, 1)
    out_tc = _tc_call(relp, rt, h[SB:], t[SB:]).reshape(MT)
    return jnp.concatenate([out_sc, out_tc])


# speedup vs baseline: 1.1442x; 1.1442x over previous
import functools

import jax
import jax.numpy as jnp
from jax import lax
from jax.experimental import pallas as pl
from jax.experimental.pallas import tpu as pltpu
from jax.experimental.pallas import tpu_sc as plsc

B = 16384
D = 128
L = 16
NC = 2
NS = 16
NW = NC * NS
SB = 4096
BPW = SB // NW
CH = 128
NCHUNK = BPW // CH
NREL = 1000
NRELP = 1024
BLK = 512
SBB = SB // BLK
MT = B - SB
NB = MT // BLK
NBALL = B // BLK

_mesh = plsc.VectorSubcoreMesh(core_axis_name="c", subcore_axis_name="s")


@functools.partial(
    pl.kernel,
    out_type=jax.ShapeDtypeStruct((SB,), jnp.float32),
    mesh=_mesh,
    compiler_params=pltpu.CompilerParams(needs_layout_passes=False),
    scratch_types=[
        pltpu.VMEM((BPW,), jnp.int32),
        pltpu.VMEM((BPW,), jnp.float32),
        pltpu.VMEM((2, CH, D), jnp.float32),
        pltpu.VMEM((2, CH, D), jnp.float32),
        pltpu.VMEM((2, CH, D), jnp.float32),
        pltpu.SemaphoreType.DMA,
        pltpu.SemaphoreType.DMA,
    ],
)
def _distmult_sc(h_hbm, r_hbm, t_hbm, rel_hbm, out_hbm,
                 idx_v, o_v, h_b, t_b, rel_b, sem0, sem1):
    wid = lax.axis_index("s") * NC + lax.axis_index("c")
    base = wid * BPW

    lane = lax.iota(jnp.int32, L)
    perms = [lane ^ s for s in (8, 4, 2, 1)]
    lane0 = lane == 0
    sems = (sem0, sem1)

    def start_ht(c):
        k = c & 1
        cbase = base + c * CH
        return (
            pltpu.async_copy(h_hbm.at[pl.ds(cbase, CH), :], h_b.at[k], sems[k]),
            pltpu.async_copy(t_hbm.at[pl.ds(cbase, CH), :], t_b.at[k], sems[k]),
        )

    def start_g(c):
        k = c & 1
        return (
            pltpu.async_copy(rel_hbm.at[idx_v.at[pl.ds(c * CH, CH)]],
                             rel_b.at[k], sems[k]),
        )

    pend = start_ht(0)
    pltpu.sync_copy(r_hbm.at[pl.ds(base, BPW)], idx_v)
    pend = pend + start_g(0)

    for c in range(NCHUNK):
        nxt = (start_ht(c + 1) + start_g(c + 1)) if c + 1 < NCHUNK else None
        for dsc in pend:
            dsc.wait()
        k = c & 1
        hk, tk, rk = h_b.at[k], t_b.at[k], rel_b.at[k]
        obase = c * CH

        def row(i, _):
            acc = hk[i, pl.ds(0, L)] * rk[i, pl.ds(0, L)] * tk[i, pl.ds(0, L)]
            for j in range(1, D // L):
                sl = pl.ds(j * L, L)
                acc = acc + hk[i, sl] * rk[i, sl] * tk[i, sl]
            for pm in perms:
                acc = acc + acc.at[pm].get(mode="promise_in_bounds")
            plsc.store_scatter(o_v, [jnp.full((L,), obase + i, jnp.int32)],
                               acc, mask=lane0)
            return 0

        lax.fori_loop(0, CH, row, 0, unroll=2)
        pend = nxt

    pltpu.sync_copy(o_v, out_hbm.at[pl.ds(base, BPW)])


def _tc_body(rel_ref, r_ref, h_ref, t_ref, out_ref):
    r2 = r_ref[0]
    k_iota = lax.broadcasted_iota(jnp.int32, (BLK, NRELP), 1)
    oh = (k_iota == r2).astype(jnp.bfloat16)
    g = lax.dot_general(oh, rel_ref[...], (((1,), (0,)), ((), ())),
                        preferred_element_type=jnp.float32)
    s = jnp.sum(h_ref[...] * g * t_ref[...], axis=1, keepdims=True)
    out_ref[...] = s


_tc_call = pl.pallas_call(
    _tc_body,
    grid=(NB,),
    in_specs=[
        pl.BlockSpec((NRELP, D), lambda i: (0, 0)),
        pl.BlockSpec((1, BLK, 1), lambda i: (i + SBB, 0, 0)),
        pl.BlockSpec((BLK, D), lambda i: (i + SBB, 0)),
        pl.BlockSpec((BLK, D), lambda i: (i + SBB, 0)),
    ],
    out_specs=pl.BlockSpec((BLK, 1), lambda i: (i, 0)),
    out_shape=jax.ShapeDtypeStruct((MT, 1), jnp.float32),
)


def kernel(h, r, t, mode, rel_emb):
    del mode
    r32 = r.astype(jnp.int32)
    out_sc = _distmult_sc(h, r32, t, rel_emb)
    relp = jnp.zeros((NRELP, D), jnp.bfloat16).at[:NREL].set(
        rel_emb.astype(jnp.bfloat16))
    rt = r32.reshape(NBALL, BLK, 1)
    out_tc = _tc_call(relp, rt, h, t).reshape(MT)
    return jnp.concatenate([out_sc, out_tc])


# speedup vs baseline: 1.5277x; 1.3351x over previous
import functools

import jax
import jax.numpy as jnp
from jax import lax
from jax.experimental import pallas as pl
from jax.experimental.pallas import tpu as pltpu
from jax.experimental.pallas import tpu_sc as plsc

B = 16384
D = 128
L = 16
NC = 2
NS = 16
NW = NC * NS
SB = 4096
BPW = SB // NW
CH = 128
NCHUNK = BPW // CH
NREL = 1000
NRELP = 1024
BLK = 512
SBB = SB // BLK
MT = B - SB
NB = MT // BLK
NBALL = B // BLK

_mesh = plsc.VectorSubcoreMesh(core_axis_name="c", subcore_axis_name="s")


@functools.partial(
    pl.kernel,
    out_type=jax.ShapeDtypeStruct((SB,), jnp.float32),
    mesh=_mesh,
    compiler_params=pltpu.CompilerParams(needs_layout_passes=False),
    scratch_types=[
        pltpu.VMEM((BPW,), jnp.int32),
        pltpu.VMEM((BPW,), jnp.float32),
        pltpu.VMEM((2, CH, D), jnp.float32),
        pltpu.VMEM((2, CH, D), jnp.float32),
        pltpu.VMEM((2, CH, D), jnp.float32),
        pltpu.SemaphoreType.DMA,
        pltpu.SemaphoreType.DMA,
    ],
)
def _distmult_sc(h_hbm, r_hbm, t_hbm, rel_hbm, out_hbm,
                 idx_v, o_v, h_b, t_b, rel_b, sem0, sem1):
    wid = lax.axis_index("s") * NC + lax.axis_index("c")
    base = wid * BPW

    lane = lax.iota(jnp.int32, L)
    perms = [lane ^ s for s in (8, 4, 2, 1)]
    lane0 = lane == 0
    sems = (sem0, sem1)

    def start_ht(c):
        k = c & 1
        cbase = base + c * CH
        return (
            pltpu.async_copy(h_hbm.at[pl.ds(cbase, CH), :], h_b.at[k], sems[k]),
            pltpu.async_copy(t_hbm.at[pl.ds(cbase, CH), :], t_b.at[k], sems[k]),
        )

    def start_g(c):
        k = c & 1
        return (
            pltpu.async_copy(rel_hbm.at[idx_v.at[pl.ds(c * CH, CH)]],
                             rel_b.at[k], sems[k]),
        )

    pend = start_ht(0)
    pltpu.sync_copy(r_hbm.at[pl.ds(base, BPW)], idx_v)
    pend = pend + start_g(0)

    for c in range(NCHUNK):
        nxt = (start_ht(c + 1) + start_g(c + 1)) if c + 1 < NCHUNK else None
        for dsc in pend:
            dsc.wait()
        k = c & 1
        hk, tk, rk = h_b.at[k], t_b.at[k], rel_b.at[k]
        obase = c * CH

        def row(i, _):
            acc = hk[i, pl.ds(0, L)] * rk[i, pl.ds(0, L)] * tk[i, pl.ds(0, L)]
            for j in range(1, D // L):
                sl = pl.ds(j * L, L)
                acc = acc + hk[i, sl] * rk[i, sl] * tk[i, sl]
            for pm in perms:
                acc = acc + acc.at[pm].get(mode="promise_in_bounds")
            plsc.store_scatter(o_v, [jnp.full((L,), obase + i, jnp.int32)],
                               acc, mask=lane0)
            return 0

        lax.fori_loop(0, CH, row, 0, unroll=2)
        pend = nxt

    pltpu.sync_copy(o_v, out_hbm.at[pl.ds(base, BPW)])


def _tc_body(rel_ref, r_ref, h_ref, t_ref, out_ref):
    r2 = r_ref[0]
    k_iota = lax.broadcasted_iota(jnp.int32, (NRELP, BLK), 0)
    oh_t = (k_iota == r2).astype(jnp.bfloat16)
    g = lax.dot_general(oh_t, rel_ref[...], (((0,), (0,)), ((), ())),
                        preferred_element_type=jnp.float32)
    prod = h_ref[...] * g * t_ref[...]
    ones = jnp.ones((1, D), jnp.float32)
    s = lax.dot_general(ones, prod, (((1,), (1,)), ((), ())),
                        preferred_element_type=jnp.float32)
    out_ref[...] = s[None]


_tc_call = pl.pallas_call(
    _tc_body,
    grid=(NB,),
    in_specs=[
        pl.BlockSpec((NRELP, D), lambda i: (0, 0)),
        pl.BlockSpec((1, 1, BLK), lambda i: (i + SBB, 0, 0)),
        pl.BlockSpec((BLK, D), lambda i: (i + SBB, 0)),
        pl.BlockSpec((BLK, D), lambda i: (i + SBB, 0)),
    ],
    out_specs=pl.BlockSpec((1, 1, BLK), lambda i: (i, 0, 0)),
    out_shape=jax.ShapeDtypeStruct((NB, 1, BLK), jnp.float32),
)


def kernel(h, r, t, mode, rel_emb):
    del mode
    r32 = r.astype(jnp.int32)
    out_sc = _distmult_sc(h, r32, t, rel_emb)
    relp = jnp.zeros((NRELP, D), jnp.bfloat16).at[:NREL].set(
        rel_emb.astype(jnp.bfloat16))
    rt = r32.reshape(NBALL, 1, BLK)
    out_tc = _tc_call(relp, rt, h, t).reshape(MT)
    return jnp.concatenate([out_sc, out_tc])


# speedup vs baseline: 1.7337x; 1.1349x over previous
import functools

import jax
import jax.numpy as jnp
from jax import lax
from jax.experimental import pallas as pl
from jax.experimental.pallas import tpu as pltpu
from jax.experimental.pallas import tpu_sc as plsc

B = 16384
D = 128
L = 16
NC = 2
NS = 16
NW = NC * NS
SB = 4096
BPW = SB // NW
CH = 128
NCHUNK = BPW // CH
NREL = 1000
NRELP = 1024
BLK = 1024
SBB = SB // BLK
MT = B - SB
NB = MT // BLK
NBALL = B // BLK

_mesh = plsc.VectorSubcoreMesh(core_axis_name="c", subcore_axis_name="s")


@functools.partial(
    pl.kernel,
    out_type=jax.ShapeDtypeStruct((SB,), jnp.float32),
    mesh=_mesh,
    compiler_params=pltpu.CompilerParams(needs_layout_passes=False),
    scratch_types=[
        pltpu.VMEM((BPW,), jnp.int32),
        pltpu.VMEM((BPW,), jnp.float32),
        pltpu.VMEM((2, CH, D), jnp.float32),
        pltpu.VMEM((2, CH, D), jnp.float32),
        pltpu.VMEM((2, CH, D), jnp.float32),
        pltpu.SemaphoreType.DMA,
        pltpu.SemaphoreType.DMA,
    ],
)
def _distmult_sc(h_hbm, r_hbm, t_hbm, rel_hbm, out_hbm,
                 idx_v, o_v, h_b, t_b, rel_b, sem0, sem1):
    wid = lax.axis_index("s") * NC + lax.axis_index("c")
    base = wid * BPW

    lane = lax.iota(jnp.int32, L)
    perms = [lane ^ s for s in (8, 4, 2, 1)]
    lane0 = lane == 0
    sems = (sem0, sem1)

    def start_ht(c):
        k = c & 1
        cbase = base + c * CH
        return (
            pltpu.async_copy(h_hbm.at[pl.ds(cbase, CH), :], h_b.at[k], sems[k]),
            pltpu.async_copy(t_hbm.at[pl.ds(cbase, CH), :], t_b.at[k], sems[k]),
        )

    def start_g(c):
        k = c & 1
        return (
            pltpu.async_copy(rel_hbm.at[idx_v.at[pl.ds(c * CH, CH)]],
                             rel_b.at[k], sems[k]),
        )

    pend = start_ht(0)
    pltpu.sync_copy(r_hbm.at[pl.ds(base, BPW)], idx_v)
    pend = pend + start_g(0)

    for c in range(NCHUNK):
        nxt = (start_ht(c + 1) + start_g(c + 1)) if c + 1 < NCHUNK else None
        for dsc in pend:
            dsc.wait()
        k = c & 1
        hk, tk, rk = h_b.at[k], t_b.at[k], rel_b.at[k]
        obase = c * CH

        def row(i, _):
            acc = hk[i, pl.ds(0, L)] * rk[i, pl.ds(0, L)] * tk[i, pl.ds(0, L)]
            for j in range(1, D // L):
                sl = pl.ds(j * L, L)
                acc = acc + hk[i, sl] * rk[i, sl] * tk[i, sl]
            for pm in perms:
                acc = acc + acc.at[pm].get(mode="promise_in_bounds")
            plsc.store_scatter(o_v, [jnp.full((L,), obase + i, jnp.int32)],
                               acc, mask=lane0)
            return 0

        lax.fori_loop(0, CH, row, 0, unroll=2)
        pend = nxt

    pltpu.sync_copy(o_v, out_hbm.at[pl.ds(base, BPW)])


def _tc_body(rel_ref, r_ref, h_ref, t_ref, out_ref):
    r2 = r_ref[0].astype(jnp.int16)
    k_iota = lax.broadcasted_iota(jnp.int16, (NRELP, BLK), 0)
    oh_t = (k_iota == r2).astype(jnp.bfloat16)
    g = lax.dot_general(oh_t, rel_ref[...], (((0,), (0,)), ((), ())),
                        preferred_element_type=jnp.float32)
    prod = h_ref[...] * g * t_ref[...]
    ones = jnp.ones((1, D), jnp.float32)
    s = lax.dot_general(ones, prod, (((1,), (1,)), ((), ())),
                        preferred_element_type=jnp.float32)
    out_ref[...] = s[None]


_tc_call = pl.pallas_call(
    _tc_body,
    grid=(NB,),
    in_specs=[
        pl.BlockSpec((NRELP, D), lambda i: (0, 0)),
        pl.BlockSpec((1, 1, BLK), lambda i: (i + SBB, 0, 0)),
        pl.BlockSpec((BLK, D), lambda i: (i + SBB, 0)),
        pl.BlockSpec((BLK, D), lambda i: (i + SBB, 0)),
    ],
    out_specs=pl.BlockSpec((1, 1, BLK), lambda i: (i, 0, 0)),
    out_shape=jax.ShapeDtypeStruct((NB, 1, BLK), jnp.float32),
)


def kernel(h, r, t, mode, rel_emb):
    del mode
    r32 = r.astype(jnp.int32)
    out_sc = _distmult_sc(h, r32, t, rel_emb)
    relp = jnp.zeros((NRELP, D), jnp.bfloat16).at[:NREL].set(
        rel_emb.astype(jnp.bfloat16))
    rt = r32.reshape(NBALL, 1, BLK)
    out_tc = _tc_call(relp, rt, h, t).reshape(MT)
    return jnp.concatenate([out_sc, out_tc])


# speedup vs baseline: 1.8546x; 1.0697x over previous
import functools

import jax
import jax.numpy as jnp
from jax import lax
from jax.experimental import pallas as pl
from jax.experimental.pallas import tpu as pltpu
from jax.experimental.pallas import tpu_sc as plsc

B = 16384
D = 128
L = 16
NC = 2
NS = 16
NW = NC * NS
SB = 6144
BPW = SB // NW
CH = 64
NCHUNK = BPW // CH
NREL = 1000
NRELP = 1024
BLK = 1024
SBB = SB // BLK
MT = B - SB
NB = MT // BLK
NBALL = B // BLK

_mesh = plsc.VectorSubcoreMesh(core_axis_name="c", subcore_axis_name="s")


@functools.partial(
    pl.kernel,
    out_type=jax.ShapeDtypeStruct((SB,), jnp.float32),
    mesh=_mesh,
    compiler_params=pltpu.CompilerParams(needs_layout_passes=False),
    scratch_types=[
        pltpu.VMEM((BPW,), jnp.int32),
        pltpu.VMEM((BPW,), jnp.float32),
        pltpu.VMEM((2, CH, D), jnp.float32),
        pltpu.VMEM((2, CH, D), jnp.float32),
        pltpu.VMEM((2, CH, D), jnp.float32),
        pltpu.SemaphoreType.DMA,
        pltpu.SemaphoreType.DMA,
    ],
)
def _distmult_sc(h_hbm, r_hbm, t_hbm, rel_hbm, out_hbm,
                 idx_v, o_v, h_b, t_b, rel_b, sem0, sem1):
    wid = lax.axis_index("s") * NC + lax.axis_index("c")
    base = wid * BPW

    lane = lax.iota(jnp.int32, L)
    perms = [lane ^ s for s in (8, 4, 2, 1)]
    lane0 = lane == 0
    sems = (sem0, sem1)

    def start_ht(c):
        k = c & 1
        cbase = base + c * CH
        return (
            pltpu.async_copy(h_hbm.at[pl.ds(cbase, CH), :], h_b.at[k], sems[k]),
            pltpu.async_copy(t_hbm.at[pl.ds(cbase, CH), :], t_b.at[k], sems[k]),
        )

    def start_g(c):
        k = c & 1
        return (
            pltpu.async_copy(rel_hbm.at[idx_v.at[pl.ds(c * CH, CH)]],
                             rel_b.at[k], sems[k]),
        )

    pend = start_ht(0)
    pltpu.sync_copy(r_hbm.at[pl.ds(base, BPW)], idx_v)
    pend = pend + start_g(0)

    for c in range(NCHUNK):
        nxt = (start_ht(c + 1) + start_g(c + 1)) if c + 1 < NCHUNK else None
        for dsc in pend:
            dsc.wait()
        k = c & 1
        hk, tk, rk = h_b.at[k], t_b.at[k], rel_b.at[k]
        obase = c * CH

        def row(i, _):
            acc = hk[i, pl.ds(0, L)] * rk[i, pl.ds(0, L)] * tk[i, pl.ds(0, L)]
            for j in range(1, D // L):
                sl = pl.ds(j * L, L)
                acc = acc + hk[i, sl] * rk[i, sl] * tk[i, sl]
            for pm in perms:
                acc = acc + acc.at[pm].get(mode="promise_in_bounds")
            plsc.store_scatter(o_v, [jnp.full((L,), obase + i, jnp.int32)],
                               acc, mask=lane0)
            return 0

        lax.fori_loop(0, CH, row, 0, unroll=2)
        pend = nxt

    pltpu.sync_copy(o_v, out_hbm.at[pl.ds(base, BPW)])


def _tc_body(rel_ref, r_ref, h_ref, t_ref, out_ref):
    r2 = r_ref[0].astype(jnp.int16)
    k_iota = lax.broadcasted_iota(jnp.int16, (NRELP, BLK), 0)
    oh_t = (k_iota == r2).astype(jnp.bfloat16)
    g = lax.dot_general(oh_t, rel_ref[...], (((0,), (0,)), ((), ())),
                        preferred_element_type=jnp.float32)
    prod = h_ref[...] * g * t_ref[...]
    ones = jnp.ones((1, D), jnp.float32)
    s = lax.dot_general(ones, prod, (((1,), (1,)), ((), ())),
                        preferred_element_type=jnp.float32)
    out_ref[...] = s[None]


_tc_call = pl.pallas_call(
    _tc_body,
    grid=(NB,),
    in_specs=[
        pl.BlockSpec((NRELP, D), lambda i: (0, 0)),
        pl.BlockSpec((1, 1, BLK), lambda i: (i + SBB, 0, 0)),
        pl.BlockSpec((BLK, D), lambda i: (i + SBB, 0)),
        pl.BlockSpec((BLK, D), lambda i: (i + SBB, 0)),
    ],
    out_specs=pl.BlockSpec((1, 1, BLK), lambda i: (i, 0, 0)),
    out_shape=jax.ShapeDtypeStruct((NB, 1, BLK), jnp.float32),
)


def kernel(h, r, t, mode, rel_emb):
    del mode
    r32 = r.astype(jnp.int32)
    out_sc = _distmult_sc(h, r32, t, rel_emb)
    relp = jnp.zeros((NRELP, D), jnp.bfloat16).at[:NREL].set(
        rel_emb.astype(jnp.bfloat16))
    rt = r32.reshape(NBALL, 1, BLK)
    out_tc = _tc_call(relp, rt, h, t).reshape(MT)
    return jnp.concatenate([out_sc, out_tc])
